# baseline (speedup 1.0000x reference)
import jax
import jax.numpy as jnp
from jax.experimental import pallas as pl

K_CB = 1024
D_CB = 256


def _vq_kernel(x_ref, cb_ref, out_ref):
    x = x_ref[0].reshape(D_CB, -1)
    cb = cb_ref[...]
    mm2 = jnp.dot(cb * -2.0, x, preferred_element_type=jnp.float32)
    cnorm = jnp.sum(cb * cb, axis=1, keepdims=True)
    fnorm = jnp.sum(x * x, axis=0, keepdims=True)
    dist = (fnorm + cnorm) + mm2
    minv = jnp.min(dist, axis=0, keepdims=True)
    kio = jax.lax.broadcasted_iota(jnp.int32, dist.shape, 0)
    idx = jnp.min(jnp.where(dist == minv, kio, K_CB), axis=0)
    out_ref[0, 0, :] = idx.astype(jnp.int32)


def kernel(z_e_x, embedding_weight):
    B, D, H, W = z_e_x.shape
    hw = H * W
    out = pl.pallas_call(
        _vq_kernel,
        grid=(B,),
        in_specs=[
            pl.BlockSpec((1, D, H, W), lambda b: (b, 0, 0, 0)),
            pl.BlockSpec((K_CB, D_CB), lambda b: (0, 0)),
        ],
        out_specs=pl.BlockSpec((1, 1, hw), lambda b: (b, 0, 0)),
        out_shape=jax.ShapeDtypeStruct((B, 1, hw), jnp.int32),
    )(z_e_x, embedding_weight)
    return out.reshape(B, H, W)


# speedup vs baseline: 1.8600x; 1.8600x over previous
import jax
import jax.numpy as jnp
from jax.experimental import pallas as pl

K_CB = 1024
D_CB = 256
NB = 1024


def _vq_kernel(f_ref, cb_ref, out_ref):
    f = f_ref[...]
    cb = cb_ref[...]
    dn = (((1,), (1,)), ((), ()))
    mm2 = jax.lax.dot_general(f, cb * -2.0, dn,
                              preferred_element_type=jnp.float32)
    ones = jnp.ones((1, D_CB), jnp.float32)
    cnorm = jax.lax.dot_general(ones, cb * cb, dn,
                                precision=jax.lax.Precision.HIGHEST,
                                preferred_element_type=jnp.float32)
    fnorm = jnp.sum(f * f, axis=1, keepdims=True)
    dist = (fnorm + cnorm) + mm2
    minv = jnp.min(dist, axis=1, keepdims=True)
    kio = jax.lax.broadcasted_iota(jnp.int32, dist.shape, 1)
    idx = jnp.min(jnp.where(dist == minv, kio, K_CB), axis=1)
    out_ref[:, 0] = idx.astype(jnp.int32)


def kernel(z_e_x, embedding_weight):
    B, D, H, W = z_e_x.shape
    n = B * H * W
    flat = jnp.transpose(z_e_x, (0, 2, 3, 1)).reshape(n, D)
    out = pl.pallas_call(
        _vq_kernel,
        grid=(n // NB,),
        in_specs=[
            pl.BlockSpec((NB, D_CB), lambda i: (i, 0)),
            pl.BlockSpec((K_CB, D_CB), lambda i: (0, 0)),
        ],
        out_specs=pl.BlockSpec((NB, 1), lambda i: (i, 0)),
        out_shape=jax.ShapeDtypeStruct((n, 1), jnp.int32),
    )(flat, embedding_weight)
    return out.reshape(B, H, W)


# speedup vs baseline: 1.9626x; 1.0552x over previous
import jax
import jax.numpy as jnp
from jax.experimental import pallas as pl

K_CB = 1024
D_CB = 256
NB = 1024


def _vq_kernel(f_ref, cb_ref, out_ref):
    f = f_ref[...]
    cb = cb_ref[...]
    dn = (((1,), (1,)), ((), ()))
    mm2 = jax.lax.dot_general(cb * -2.0, f, dn,
                              preferred_element_type=jnp.float32)
    ones = jnp.ones((1, D_CB), jnp.float32)
    fnorm = jax.lax.dot_general(ones, f * f, dn,
                                precision=jax.lax.Precision.HIGHEST,
                                preferred_element_type=jnp.float32)
    cnorm = jnp.sum(cb * cb, axis=1, keepdims=True)
    dist = (fnorm + cnorm) + mm2
    minv = jnp.min(dist, axis=0, keepdims=True)
    kio = jax.lax.broadcasted_iota(jnp.int32, dist.shape, 0)
    idx = jnp.min(jnp.where(dist == minv, kio, K_CB), axis=0)
    out_ref[0, 0, :] = idx.astype(jnp.int32)


def kernel(z_e_x, embedding_weight):
    B, D, H, W = z_e_x.shape
    n = B * H * W
    flat = jnp.transpose(z_e_x, (0, 2, 3, 1)).reshape(n, D)
    out = pl.pallas_call(
        _vq_kernel,
        grid=(n // NB,),
        in_specs=[
            pl.BlockSpec((NB, D_CB), lambda i: (i, 0)),
            pl.BlockSpec((K_CB, D_CB), lambda i: (0, 0)),
        ],
        out_specs=pl.BlockSpec((1, 1, NB), lambda i: (i, 0, 0)),
        out_shape=jax.ShapeDtypeStruct((n // NB, 1, NB), jnp.int32),
    )(flat, embedding_weight)
    return out.reshape(B, H, W)


# speedup vs baseline: 2.6191x; 1.3345x over previous
import jax
import jax.numpy as jnp
from jax.experimental import pallas as pl

K_CB = 1024
D_CB = 256
NB = 2048


def _vq_kernel(f_ref, cb_ref, out_ref):
    f = f_ref[...]
    cb = cb_ref[...]
    dn = (((1,), (1,)), ((), ()))
    mm2 = jax.lax.dot_general(cb * -2.0, f, dn,
                              preferred_element_type=jnp.float32)
    ones = jnp.ones((1, D_CB), jnp.float32)
    g = f * f
    g_hi = g.astype(jnp.bfloat16)
    g_lo = (g - g_hi.astype(jnp.float32)).astype(jnp.bfloat16)
    fnorm = (jax.lax.dot_general(ones, g_hi, dn,
                                 preferred_element_type=jnp.float32)
             + jax.lax.dot_general(ones, g_lo, dn,
                                   preferred_element_type=jnp.float32))
    cnorm = jnp.sum(cb * cb, axis=1, keepdims=True)
    dist = (fnorm + cnorm) + mm2
    minv = jnp.min(dist, axis=0, keepdims=True)
    kio = jax.lax.broadcasted_iota(jnp.int32, dist.shape, 0)
    fio = kio.astype(jnp.float32)
    idx = jnp.min(jnp.where(dist == minv, fio, float(K_CB)), axis=0)
    out_ref[0, 0, :] = idx.astype(jnp.int32)


def kernel(z_e_x, embedding_weight):
    B, D, H, W = z_e_x.shape
    n = B * H * W
    flat = jnp.transpose(z_e_x, (0, 2, 3, 1)).reshape(n, D)
    out = pl.pallas_call(
        _vq_kernel,
        grid=(n // NB,),
        in_specs=[
            pl.BlockSpec((NB, D_CB), lambda i: (i, 0)),
            pl.BlockSpec((K_CB, D_CB), lambda i: (0, 0)),
        ],
        out_specs=pl.BlockSpec((1, 1, NB), lambda i: (i, 0, 0)),
        out_shape=jax.ShapeDtypeStruct((n // NB, 1, NB), jnp.int32),
    )(flat, embedding_weight)
    return out.reshape(B, H, W)
